# baseline (speedup 1.0000x reference)
import functools

import jax
import jax.numpy as jnp
from jax import lax
from jax.experimental import pallas as pl
from jax.experimental.pallas import tpu as pltpu
from jax.experimental.pallas import tpu_sc as plsc

N_ROWS = 128
N_CLS = 4096
NC = 2
NS = 16
L = 16
NW = NC * NS
RPW = N_ROWS // NW
VPR = N_CLS // L


def _st_one_hot_body(logits_hbm, hard_hbm, probs_hbm, xbuf, hardbuf):
    wid = lax.axis_index("s") * NC + lax.axis_index("c")
    base = wid * RPW
    pltpu.sync_copy(logits_hbm.at[pl.ds(base, RPW)], xbuf)

    lane = lax.iota(jnp.int32, L)

    for r in range(RPW):
        def amax_body(j, m):
            off = pl.multiple_of(j * L, L)
            return jnp.maximum(m, xbuf[r, pl.ds(off, L)])

        m = lax.fori_loop(0, VPR, amax_body,
                          jnp.full((L,), -jnp.inf, jnp.float32))
        m = jnp.max(m)

        def exp_body(j, carry):
            s, bv, bi = carry
            off = pl.multiple_of(j * L, L)
            e = jnp.exp(xbuf[r, pl.ds(off, L)] - m)
            xbuf[r, pl.ds(off, L)] = e
            gidx = j * L + lane
            upd = e > bv
            return (s + e,
                    jnp.where(upd, e, bv),
                    jnp.where(upd, gidx, bi))

        s, bv, bi = lax.fori_loop(
            0, VPR, exp_body,
            (jnp.zeros((L,), jnp.float32),
             jnp.full((L,), -jnp.inf, jnp.float32),
             jnp.zeros((L,), jnp.int32)))
        stot = jnp.sum(s)
        mval = jnp.max(bv)
        best = jnp.min(jnp.where(bv == mval, bi, jnp.int32(2**31 - 1)))

        def norm_body(j, _):
            off = pl.multiple_of(j * L, L)
            e = xbuf[r, pl.ds(off, L)]
            xbuf[r, pl.ds(off, L)] = e / stot
            gidx = j * L + lane
            hardbuf[r, pl.ds(off, L)] = jnp.where(
                gidx == best, jnp.float32(1.0), jnp.float32(0.0))
            return 0

        lax.fori_loop(0, VPR, norm_body, 0)

    pltpu.sync_copy(hardbuf, hard_hbm.at[pl.ds(base, RPW)])
    pltpu.sync_copy(xbuf, probs_hbm.at[pl.ds(base, RPW)])


@jax.jit
def kernel(logits):
    out_sds = jax.ShapeDtypeStruct((N_ROWS, N_CLS), jnp.float32)
    mesh = plsc.VectorSubcoreMesh(core_axis_name="c", subcore_axis_name="s",
                                  num_cores=NC, num_subcores=NS)
    hard, probs = pl.kernel(
        _st_one_hot_body,
        out_type=(out_sds, out_sds),
        mesh=mesh,
        scratch_types=[
            pltpu.VMEM((RPW, N_CLS), jnp.float32),
            pltpu.VMEM((RPW, N_CLS), jnp.float32),
        ],
        compiler_params=pltpu.CompilerParams(needs_layout_passes=False),
    )(logits)
    return (hard, probs)


# speedup vs baseline: 1.5472x; 1.5472x over previous
import functools

import jax
import jax.numpy as jnp
from jax import lax
from jax.experimental import pallas as pl
from jax.experimental.pallas import tpu as pltpu
from jax.experimental.pallas import tpu_sc as plsc

N_ROWS = 128
N_CLS = 4096
NC = 2
NS = 16
L = 16
NW = NC * NS
RPW = N_ROWS // NW
UNROLL = 8


def _st_one_hot_body(logits_hbm, hard_hbm, probs_hbm, xbuf, hardbuf, sem):
    wid = lax.axis_index("s") * NC + lax.axis_index("c")
    base = wid * RPW
    pltpu.sync_copy(logits_hbm.at[pl.ds(base, RPW)], xbuf)

    lane = lax.iota(jnp.int32, L)
    copies = []

    for r in range(RPW):
        @plsc.parallel_loop(0, N_CLS, step=L, unroll=UNROLL,
                            carry=jnp.full((L,), -jnp.inf, jnp.float32))
        def row_max(i, m):
            return jnp.maximum(m, xbuf[r, pl.ds(i, L)])

        m = jnp.max(row_max)

        @plsc.parallel_loop(
            0, N_CLS, step=L, unroll=UNROLL,
            carry=(jnp.zeros((L,), jnp.float32),
                   jnp.full((L,), -jnp.inf, jnp.float32),
                   jnp.full((L,), 2**31 - 1, jnp.int32)))
        def exp_sum(i, carry):
            s, bv, bi = carry
            e = jnp.exp(xbuf[r, pl.ds(i, L)] - m)
            xbuf[r, pl.ds(i, L)] = e
            gidx = i + lane
            upd = (e > bv) | ((e == bv) & (gidx < bi))
            return (s + e,
                    jnp.where(upd, e, bv),
                    jnp.where(upd, gidx, bi))

        s, bv, bi = exp_sum
        rinv = jnp.full((L,), 1.0, jnp.float32) / jnp.sum(s)
        mval = jnp.max(bv)
        best = jnp.min(jnp.where(bv == mval, bi, jnp.int32(2**31 - 1)))

        @plsc.parallel_loop(0, N_CLS, step=L, unroll=UNROLL)
        def norm(i):
            e = xbuf[r, pl.ds(i, L)]
            xbuf[r, pl.ds(i, L)] = e * rinv
            gidx = i + lane
            hardbuf[r, pl.ds(i, L)] = jnp.where(
                gidx == best, jnp.float32(1.0), jnp.float32(0.0))

        copies.append(pltpu.async_copy(
            hardbuf.at[r], hard_hbm.at[base + r], sem))
        copies.append(pltpu.async_copy(
            xbuf.at[r], probs_hbm.at[base + r], sem))

    for c in copies:
        c.wait()


@jax.jit
def kernel(logits):
    out_sds = jax.ShapeDtypeStruct((N_ROWS, N_CLS), jnp.float32)
    mesh = plsc.VectorSubcoreMesh(core_axis_name="c", subcore_axis_name="s",
                                  num_cores=NC, num_subcores=NS)
    hard, probs = pl.kernel(
        _st_one_hot_body,
        out_type=(out_sds, out_sds),
        mesh=mesh,
        scratch_types=[
            pltpu.VMEM((RPW, N_CLS), jnp.float32),
            pltpu.VMEM((RPW, N_CLS), jnp.float32),
            pltpu.SemaphoreType.DMA,
        ],
        compiler_params=pltpu.CompilerParams(needs_layout_passes=False),
    )(logits)
    return (hard, probs)
